# Initial kernel scaffold; baseline (speedup 1.0000x reference)
import jax
import jax.numpy as jnp
from jax.experimental import pallas as pl


def kernel(x, edge_index, W1, b1, g1, be1, W2, b2, g2, be2, eps):
    raise NotImplementedError("write your pallas kernel here")


# speedup vs baseline: 6.8912x; 6.8912x over previous
import functools

import jax
import jax.numpy as jnp
from jax import lax
from jax.experimental import pallas as pl
from jax.experimental.pallas import tpu as pltpu
from jax.experimental.pallas import tpu_sc as plsc

_N, _E, _D, _L = 10000, 320000, 128, 5
_NC, _NS = 2, 16
_NW = _NC * _NS
_CH = 100
_EPT = _E // _NW
_NCH = _EPT // _CH
_NPAD = 10240
_RPT = _NPAD // _NS


def _sc_segment_sum(h, col2d, row2d, zrows):
    mesh = plsc.VectorSubcoreMesh(core_axis_name="c", subcore_axis_name="s")

    @functools.partial(
        pl.kernel,
        mesh=mesh,
        out_type=jax.ShapeDtypeStruct((_NC, _NPAD, _D), jnp.float32),
        scratch_types=[
            pltpu.VMEM((_NCH, _CH), jnp.int32),
            pltpu.VMEM((_NCH, _CH), jnp.int32),
            pltpu.VMEM((_CH, _D), jnp.float32),
            pltpu.VMEM_SHARED((_NPAD, _D), jnp.float32),
            pltpu.SemaphoreType.DMA,
        ],
    )
    def run(h_hbm, col_hbm, row_hbm, z_hbm, out_hbm,
            colv, rowv, gbuf, acc, gsem):
        c = lax.axis_index("c")
        s = lax.axis_index("s")
        tid = c * _NS + s

        pltpu.sync_copy(col_hbm.at[tid], colv)
        pltpu.sync_copy(row_hbm.at[tid], rowv)

        pltpu.sync_copy(z_hbm, acc.at[pl.ds(s * _RPT, _RPT)])
        plsc.subcore_barrier()

        def body(i, carry):
            pltpu.async_copy(h_hbm.at[colv.at[i]], gbuf, gsem).wait()
            pltpu.sync_copy(gbuf, acc.at[rowv.at[i]], add=True)
            return carry

        lax.fori_loop(0, _NCH, body, 0)
        plsc.subcore_barrier()

        pltpu.sync_copy(acc.at[pl.ds(s * _RPT, _RPT)],
                        out_hbm.at[c].at[pl.ds(s * _RPT, _RPT)])

    return run(h, col2d, row2d, zrows)


def _tc_mlp(h, p, scal, w1, b1, g1, be1, w2, b2, g2, be2):
    def body(h_ref, p_ref, sc_ref, w1_ref, b1_ref, g1_ref, be1_ref,
             w2_ref, b2_ref, g2_ref, be2_ref, out_ref):
        agg = (sc_ref[...] * h_ref[...]
               + (p_ref[0, : _N, :] + p_ref[1, : _N, :]))
        z = jnp.dot(agg, w1_ref[...], preferred_element_type=jnp.float32)
        z = z + b1_ref[...]
        mu = jnp.mean(z, axis=0, keepdims=True)
        var = jnp.mean((z - mu) ** 2, axis=0, keepdims=True)
        z = (z - mu) / jnp.sqrt(var + 1e-5) * g1_ref[...] + be1_ref[...]
        z = jnp.maximum(z, 0.0)
        z2 = jnp.dot(z, w2_ref[...], preferred_element_type=jnp.float32)
        z2 = z2 + b2_ref[...]
        mu2 = jnp.mean(z2, axis=0, keepdims=True)
        var2 = jnp.mean((z2 - mu2) ** 2, axis=0, keepdims=True)
        z2 = (z2 - mu2) / jnp.sqrt(var2 + 1e-5) * g2_ref[...] + be2_ref[...]
        out_ref[...] = jnp.maximum(z2, 0.0)

    return pl.pallas_call(
        body,
        out_shape=jax.ShapeDtypeStruct((_N, _D), jnp.float32),
    )(h, p, scal, w1, b1, g1, be1, w2, b2, g2, be2)


def kernel(x, edge_index, W1, b1, g1, be1, W2, b2, g2, be2, eps):
    row2d = edge_index[0].reshape(_NW, _NCH, _CH)
    col2d = edge_index[1].reshape(_NW, _NCH, _CH)
    zrows = jnp.zeros((_RPT, _D), jnp.float32)
    h = x
    for l in range(_L):
        p = _sc_segment_sum(h, col2d, row2d, zrows)
        scal = (1.0 + eps[l]).reshape(1, 1)
        h = _tc_mlp(h, p, scal,
                    W1[l], b1[l].reshape(1, _D), g1[l].reshape(1, _D),
                    be1[l].reshape(1, _D),
                    W2[l], b2[l].reshape(1, _D), g2[l].reshape(1, _D),
                    be2[l].reshape(1, _D))
    return h
